# baseline (speedup 1.0000x reference)
import jax
import jax.numpy as jnp
from jax import lax
from jax.experimental import pallas as pl
from jax.experimental.pallas import tpu as pltpu
from jax.experimental.pallas import tpu_sc as plsc

_NC = 2
_NS = 16
_NW = _NC * _NS
_CHUNK = 2


def _sc_body(in_hbm, emb_hbm, out_hbm, buf0, buf1, isem0, isem1, osem0, osem1):
    batch, rows, dim = in_hbm.shape
    n_emb = emb_hbm.shape[0]
    per_w = batch // _NW
    n_chunks = per_w // _CHUNK

    wid = lax.axis_index("s") * _NC + lax.axis_index("c")
    base = wid * per_w

    bufs = (buf0, buf1)
    in_sems = (isem0, isem1)
    out_sems = (osem0, osem1)

    for buf in bufs:
        for j in range(_CHUNK):
            pltpu.sync_copy(emb_hbm, buf.at[j, pl.ds(0, n_emb)])

    def in_copy(g, i):
        return pltpu.make_async_copy(
            in_hbm.at[pl.ds(base + g * _CHUNK, _CHUNK)],
            bufs[i].at[:, pl.ds(n_emb, rows)],
            in_sems[i],
        )

    def out_copy(g, i):
        return pltpu.make_async_copy(
            bufs[i],
            out_hbm.at[pl.ds(base + g * _CHUNK, _CHUNK)],
            out_sems[i],
        )

    in_copy(0, 0).start()
    out_prev = None
    for g in range(n_chunks):
        i = g % 2
        in_copy(g, i).wait()
        cur = out_copy(g, i)
        cur.start()
        if g + 1 < n_chunks:
            if out_prev is not None:
                out_prev.wait()
            in_copy(g + 1, (g + 1) % 2).start()
        out_prev = cur
    out_prev.wait()


@jax.jit
def kernel(inputs, emb_table):
    batch, rows, dim = inputs.shape
    n_emb = emb_table.shape[0]
    out_rows = rows + n_emb
    mesh = plsc.VectorSubcoreMesh(core_axis_name="c", subcore_axis_name="s")
    run = pl.kernel(
        _sc_body,
        out_type=jax.ShapeDtypeStruct((batch, out_rows, dim), inputs.dtype),
        mesh=mesh,
        scratch_types=[
            pltpu.VMEM((_CHUNK, out_rows, dim), inputs.dtype),
            pltpu.VMEM((_CHUNK, out_rows, dim), inputs.dtype),
            pltpu.SemaphoreType.DMA,
            pltpu.SemaphoreType.DMA,
            pltpu.SemaphoreType.DMA,
            pltpu.SemaphoreType.DMA,
        ],
    )
    return run(inputs, emb_table)


# speedup vs baseline: 1.0064x; 1.0064x over previous
import jax
import jax.numpy as jnp
from jax import lax
from jax.experimental import pallas as pl
from jax.experimental.pallas import tpu as pltpu
from jax.experimental.pallas import tpu_sc as plsc

_NC = 2
_NS = 16
_NW = _NC * _NS
_NBUF = 4


def _sc_body(in_hbm, emb_hbm, out_hbm, bufs, in_sems, out_sems):
    batch, rows, dim = in_hbm.shape
    n_emb = emb_hbm.shape[0]
    per_w = batch // _NW

    wid = lax.axis_index("s") * _NC + lax.axis_index("c")
    base = wid * per_w

    for i in range(_NBUF):
        pltpu.sync_copy(emb_hbm, bufs[i].at[pl.ds(0, n_emb)])

    def in_copy(g, i):
        return pltpu.make_async_copy(
            in_hbm.at[base + g],
            bufs[i].at[pl.ds(n_emb, rows)],
            in_sems[i],
        )

    def out_copy(g, i):
        return pltpu.make_async_copy(
            bufs[i],
            out_hbm.at[base + g],
            out_sems[i],
        )

    for g in range(min(_NBUF - 1, per_w)):
        in_copy(g, g % _NBUF).start()
    for g in range(per_w):
        i = g % _NBUF
        in_copy(g, i).wait()
        out_copy(g, i).start()
        nxt = g + _NBUF - 1
        if nxt < per_w:
            if g >= 1:
                out_copy(g - 1, (g - 1) % _NBUF).wait()
            in_copy(nxt, nxt % _NBUF).start()
    out_copy(per_w - 1, (per_w - 1) % _NBUF).wait()


@jax.jit
def kernel(inputs, emb_table):
    batch, rows, dim = inputs.shape
    n_emb = emb_table.shape[0]
    out_rows = rows + n_emb
    mesh = plsc.VectorSubcoreMesh(core_axis_name="c", subcore_axis_name="s")
    run = pl.kernel(
        _sc_body,
        out_type=jax.ShapeDtypeStruct((batch, out_rows, dim), inputs.dtype),
        mesh=mesh,
        scratch_types=[
            [pltpu.VMEM((out_rows, dim), inputs.dtype) for _ in range(_NBUF)],
            [pltpu.SemaphoreType.DMA for _ in range(_NBUF)],
            [pltpu.SemaphoreType.DMA for _ in range(_NBUF)],
        ],
    )
    return run(inputs, emb_table)


# speedup vs baseline: 1.2572x; 1.2493x over previous
import jax
import jax.numpy as jnp
from jax.experimental import pallas as pl
from jax.experimental.pallas import tpu as pltpu

_BBLK = 128


def _body(emb_ref, in_ref, out_ref):
    nb, ne, dim = out_ref.shape[0], emb_ref.shape[0], emb_ref.shape[1]
    out_ref[:, ne:, :] = in_ref[...]
    out_ref[:, :ne, :] = jnp.broadcast_to(emb_ref[...][None, :, :], (nb, ne, dim))


@jax.jit
def kernel(inputs, emb_table):
    batch, rows, dim = inputs.shape
    n_emb = emb_table.shape[0]
    out_shape = jax.ShapeDtypeStruct((batch, rows + n_emb, dim), inputs.dtype)
    grid = (batch // _BBLK,)
    return pl.pallas_call(
        _body,
        out_shape=out_shape,
        grid=grid,
        in_specs=[
            pl.BlockSpec((n_emb, dim), lambda b: (0, 0)),
            pl.BlockSpec((_BBLK, rows, dim), lambda b: (b, 0, 0)),
        ],
        out_specs=pl.BlockSpec((_BBLK, rows + n_emb, dim), lambda b: (b, 0, 0)),
    )(emb_table, inputs)
